# baseline (speedup 1.0000x reference)
import functools

import jax
import jax.numpy as jnp
from jax import lax
from jax.experimental import pallas as pl
from jax.experimental.pallas import tpu as pltpu
from jax.experimental.pallas import tpu_sc as plsc

_NC = 2
_NS = 16
_NW = _NC * _NS
_CH = 128


@functools.lru_cache(maxsize=None)
def _build_gather(n_rows: int, d: int):
    assert n_rows % (_NW * _CH) == 0
    nch = n_rows // (_NW * _CH)
    mesh = plsc.VectorSubcoreMesh(core_axis_name="c", subcore_axis_name="s")

    @functools.partial(
        pl.kernel,
        mesh=mesh,
        out_type=jax.ShapeDtypeStruct((n_rows, d), jnp.float32),
        scratch_types=[
            pltpu.VMEM((nch, _CH), jnp.int32),
            pltpu.VMEM((_CH, d), jnp.float32),
            pltpu.SemaphoreType.DMA,
            pltpu.SemaphoreType.DMA,
        ],
        compiler_params=pltpu.CompilerParams(use_tc_tiling_on_sc=False),
    )
    def gather(idx_hbm, table_hbm, out_hbm, idx_v, rows_v, gsem, ssem):
        wid = lax.axis_index("s") * _NC + lax.axis_index("c")
        base = wid * (nch * _CH)
        pltpu.sync_copy(idx_hbm.at[wid], idx_v)

        def body(j, carry):
            pltpu.async_copy(table_hbm.at[idx_v.at[j]], rows_v, gsem).wait()
            pltpu.async_copy(
                rows_v, out_hbm.at[pl.ds(base + j * _CH, _CH)], ssem
            ).wait()
            return carry

        lax.fori_loop(0, nch, body, 0)

    return gather


def kernel(indices, table):
    b, l = indices.shape
    _, d = table.shape
    n = b * l
    idx = indices.reshape(_NW, n // (_NW * _CH), _CH)
    rows = _build_gather(n, d)(idx, table)
    return rows.reshape(b, l, d), jnp.full((b,), l, dtype=jnp.int64)


# speedup vs baseline: 1.0466x; 1.0466x over previous
import functools

import jax
import jax.numpy as jnp
from jax import lax
from jax.experimental import pallas as pl
from jax.experimental.pallas import tpu as pltpu
from jax.experimental.pallas import tpu_sc as plsc

_NC = 2
_NS = 16
_NW = _NC * _NS
_CH = 128


_K = 5


@functools.lru_cache(maxsize=None)
def _build_gather(n_rows: int, d: int):
    assert n_rows % (_NW * _CH) == 0
    nch = n_rows // (_NW * _CH)
    nph = nch // _K
    assert nph * _K == nch and nph % 2 == 0
    rows_per_phase = _K * _CH
    mesh = plsc.VectorSubcoreMesh(core_axis_name="c", subcore_axis_name="s")

    @functools.partial(
        pl.kernel,
        mesh=mesh,
        out_type=jax.ShapeDtypeStruct((n_rows, d), jnp.float32),
        scratch_types=[
            pltpu.VMEM((nch, _CH), jnp.int32),
            pltpu.VMEM((rows_per_phase, d), jnp.float32),
            pltpu.VMEM((rows_per_phase, d), jnp.float32),
            pltpu.SemaphoreType.DMA,
            pltpu.SemaphoreType.DMA,
            pltpu.SemaphoreType.DMA,
            pltpu.SemaphoreType.DMA,
        ],
        compiler_params=pltpu.CompilerParams(use_tc_tiling_on_sc=False),
    )
    def gather(idx_hbm, table_hbm, out_hbm, idx_v, buf_a, buf_b,
               gsem_a, gsem_b, ssem_a, ssem_b):
        wid = lax.axis_index("s") * _NC + lax.axis_index("c")
        base = wid * (nch * _CH)
        pltpu.sync_copy(idx_hbm.at[wid], idx_v)

        def fire_gathers(phase, buf, sem):
            for b in range(_K):
                pltpu.async_copy(
                    table_hbm.at[idx_v.at[phase * _K + b]],
                    buf.at[pl.ds(b * _CH, _CH)],
                    sem,
                )

        def drain_gathers(phase, buf, sem):
            for b in range(_K):
                pltpu.make_async_copy(
                    table_hbm.at[idx_v.at[phase * _K + b]],
                    buf.at[pl.ds(b * _CH, _CH)],
                    sem,
                ).wait()

        def fire_scatter(phase, buf, sem):
            pltpu.async_copy(
                buf, out_hbm.at[pl.ds(base + phase * rows_per_phase,
                                      rows_per_phase)], sem)

        def drain_scatter(phase, buf, sem):
            pltpu.make_async_copy(
                buf, out_hbm.at[pl.ds(base + phase * rows_per_phase,
                                      rows_per_phase)], sem).wait()

        fire_gathers(0, buf_a, gsem_a)

        def body(i, carry):
            pa = 2 * i
            pb = 2 * i + 1

            @pl.when(i > 0)
            def _():
                drain_scatter(pb - 2, buf_b, ssem_b)

            fire_gathers(pb, buf_b, gsem_b)
            drain_gathers(pa, buf_a, gsem_a)
            fire_scatter(pa, buf_a, ssem_a)

            @pl.when(i < nph // 2 - 1)
            def _():
                drain_scatter(pa, buf_a, ssem_a)
                fire_gathers(pa + 2, buf_a, gsem_a)

            drain_gathers(pb, buf_b, gsem_b)
            fire_scatter(pb, buf_b, ssem_b)
            return carry

        lax.fori_loop(0, nph // 2, body, 0)
        drain_scatter(nch // _K - 2, buf_a, ssem_a)
        drain_scatter(nch // _K - 1, buf_b, ssem_b)

    return gather


def kernel(indices, table):
    b, l = indices.shape
    _, d = table.shape
    n = b * l
    idx = indices.reshape(_NW, n // (_NW * _CH), _CH)
    rows = _build_gather(n, d)(idx, table)
    return rows.reshape(b, l, d), jnp.full((b,), l, dtype=jnp.int64)
